# baseline (speedup 1.0000x reference)
import functools

import jax
import jax.numpy as jnp
from jax import lax
from jax.experimental import pallas as pl
from jax.experimental.pallas import tpu as pltpu
from jax.experimental.pallas import tpu_sc as plsc

F = 26
B = 4096
L = 20
D = 128
V = 100000

_info = plsc.get_sparse_core_info()
NC, NS, LANES = _info.num_cores, _info.num_subcores, _info.num_lanes
NW = NC * NS
BAGS = B * F
BPW = BAGS // NW
NG = 8
NIT = BPW // NG
NSLOT = 4
CH = 2
CHB = NG * L // CH
DCH = D // LANES
IDX_ROWS = BAGS * L // CHB


def _body(idx_hbm, tab_hbm, out_hbm, idx_v, rows_v, out_v, oidx_v, *sems):
    wid = lax.axis_index("s") * NC + lax.axis_index("c")
    idx_row0 = wid * (BPW * L // CHB)
    g0w = wid * BPW
    gsems = sems[0:NSLOT]
    isems = sems[NSLOT:2 * NSLOT]
    osems = sems[2 * NSLOT:]

    def idx_descr(it, slot):
        return pltpu.make_async_copy(
            idx_hbm.at[pl.ds(idx_row0 + it * CH, CH)],
            idx_v.at[slot],
            isems[slot],
        )

    def gather_descr(slot, j):
        return pltpu.make_async_copy(
            tab_hbm.at[idx_v.at[slot, j]],
            rows_v.at[slot, pl.ds(j * CHB, CHB)],
            gsems[slot],
        )

    def out_descr(pslot):
        return pltpu.make_async_copy(
            out_v.at[pslot],
            out_hbm.at[oidx_v.at[pslot]],
            osems[pslot],
        )

    def globalize(it, slot):
        fv = ((g0w + it * NG) // B) * V
        fvv = jnp.full((LANES,), fv, dtype=jnp.int32)
        for j in range(CH):
            for k in range(CHB // LANES):
                sl = pl.ds(k * LANES, LANES)
                idx_v[slot, j, sl] = idx_v[slot, j, sl] + fvv

    def stage(it, slot):
        idx_descr(it, slot).wait()
        globalize(it, slot)
        for j in range(CH):
            gather_descr(slot, j).start()

    def accumulate(slot, pslot, phalf):
        def bag_body(jj, c2):
            base = jj * L
            for c in range(DCH):
                v = rows_v[slot, base, pl.ds(c * LANES, LANES)]
                for l in range(1, L):
                    v = v + rows_v[slot, base + l, pl.ds(c * LANES, LANES)]
                out_v[pslot, phalf * NG + jj, pl.ds(c * LANES, LANES)] = v
            return c2

        lax.fori_loop(0, NG, bag_body, 0)

    idx_descr(0, 0).start()
    idx_descr(1, 1).start()
    stage(0, 0)
    stage(1, 1)
    idx_descr(2, 2).start()
    idx_descr(3, 3).start()

    def step(it, s):
        cur = it + s
        pslot = s // 2
        phalf = s % 2

        for j in range(CH):
            gather_descr(s, j).wait()

        @pl.when(cur + NSLOT < NIT)
        def _():
            idx_descr(cur + NSLOT, s).start()

        @pl.when(cur + 2 < NIT)
        def _():
            stage(cur + 2, (s + 2) % NSLOT)

        if phalf == 0:
            @pl.when(cur >= NSLOT)
            def _():
                out_descr(pslot).wait()

        accumulate(s, pslot, phalf)

        if phalf == 1:
            g0 = g0w + (cur - 1) * NG
            obase = (g0 % B) * F + g0 // B
            oidx_v[pslot, :] = obase + F * lax.iota(jnp.int32, LANES)
            out_descr(pslot).start()

    def it_body(i, carry):
        for s in range(NSLOT):
            step(NSLOT * i, s)
        return carry

    lax.fori_loop(0, NIT // NSLOT, it_body, 0)
    out_descr(0).wait()
    out_descr(1).wait()


_mesh = plsc.VectorSubcoreMesh(core_axis_name="c", subcore_axis_name="s")

_lookup = functools.partial(
    pl.kernel,
    mesh=_mesh,
    out_type=jax.ShapeDtypeStruct((BAGS, D), jnp.float32),
    scratch_types=[
        pltpu.VMEM((NSLOT, CH, CHB), jnp.int32),
        pltpu.VMEM((NSLOT, NG * L, D), jnp.float32),
        pltpu.VMEM((2, 2 * NG, D), jnp.float32),
        pltpu.VMEM((2, LANES), jnp.int32),
    ] + [pltpu.SemaphoreType.DMA] * (2 * NSLOT + 2),
)(_body)


def kernel(indices, tables):
    idx = indices.astype(jnp.int32).reshape(IDX_ROWS, CHB)
    tab = tables.reshape(F * V, D)
    out = _lookup(idx, tab)
    return out.reshape(B, F, D)


# speedup vs baseline: 1.5019x; 1.5019x over previous
import functools

import jax
import jax.numpy as jnp
from jax import lax
from jax.experimental import pallas as pl
from jax.experimental.pallas import tpu as pltpu
from jax.experimental.pallas import tpu_sc as plsc

F = 26
B = 4096
L = 20
D = 128
V = 100000

_info = plsc.get_sparse_core_info()
NC, NS, LANES = _info.num_cores, _info.num_subcores, _info.num_lanes
NW = NC * NS
BAGS = B * F
BPW = BAGS // NW
NG = 16
NIT = BPW // NG
CH = 4
CHB = NG * L // CH
DCH = D // LANES
IDX_ROWS = BAGS * L // CHB


def _body(idx_hbm, tab_hbm, out_hbm, idx_v, rows_v, out_v, oidx_v,
          gsem0, gsem1, osem0, osem1, isem0, isem1):
    wid = lax.axis_index("s") * NC + lax.axis_index("c")
    idx_row0 = wid * (BPW * L // CHB)
    g0w = wid * BPW
    gsems = (gsem0, gsem1)
    osems = (osem0, osem1)
    isems = (isem0, isem1)

    def idx_descr(it, slot):
        return pltpu.make_async_copy(
            idx_hbm.at[pl.ds(idx_row0 + it * CH, CH)],
            idx_v.at[slot],
            isems[slot],
        )

    def gather_descr(slot, j):
        return pltpu.make_async_copy(
            tab_hbm.at[idx_v.at[slot, j]],
            rows_v.at[slot, pl.ds(j * CHB, CHB)],
            gsems[slot],
        )

    def out_descr(slot):
        return pltpu.make_async_copy(
            out_v.at[slot],
            out_hbm.at[oidx_v.at[slot]],
            osems[slot],
        )

    def globalize(it, slot):
        fv = ((g0w + it * NG) // B) * V
        fvv = jnp.full((LANES,), fv, dtype=jnp.int32)
        for j in range(CH):
            for k in range(CHB // LANES):
                sl = pl.ds(k * LANES, LANES)
                idx_v[slot, j, sl] = idx_v[slot, j, sl] + fvv

    def accumulate(it, slot):
        def bag_body(jj, c2):
            base = jj * L
            for c in range(DCH):
                vals = [rows_v[slot, base + l, pl.ds(c * LANES, LANES)]
                        for l in range(L)]
                while len(vals) > 1:
                    nxt = [vals[i] + vals[i + 1]
                           for i in range(0, len(vals) - 1, 2)]
                    if len(vals) % 2:
                        nxt.append(vals[-1])
                    vals = nxt
                out_v[slot, jj, pl.ds(c * LANES, LANES)] = vals[0]
            return c2

        lax.fori_loop(0, NG, bag_body, 0)
        g0 = g0w + it * NG
        obase = (g0 % B) * F + g0 // B
        oidx_v[slot, :] = obase + F * lax.iota(jnp.int32, LANES)

    idx_descr(0, 0).start()
    idx_descr(0, 0).wait()
    globalize(0, 0)
    for j in range(CH):
        gather_descr(0, j).start()
    idx_descr(1, 1).start()

    def half(it, s):
        cur = it + s
        ns = 1 - s

        for j in range(CH):
            gather_descr(s, j).wait()

        @pl.when(cur + 2 < NIT)
        def _():
            idx_descr(cur + 2, s).start()

        @pl.when(cur + 1 < NIT)
        def _():
            idx_descr(cur + 1, ns).wait()
            globalize(cur + 1, ns)
            for j in range(CH):
                gather_descr(ns, j).start()

        @pl.when(cur >= 2)
        def _():
            out_descr(s).wait()

        accumulate(cur, s)
        out_descr(s).start()

    def it_body(i, carry):
        half(2 * i, 0)
        half(2 * i, 1)
        return carry

    lax.fori_loop(0, NIT // 2, it_body, 0)
    out_descr(0).wait()
    out_descr(1).wait()


_mesh = plsc.VectorSubcoreMesh(core_axis_name="c", subcore_axis_name="s")

_lookup = functools.partial(
    pl.kernel,
    mesh=_mesh,
    out_type=jax.ShapeDtypeStruct((BAGS, D), jnp.float32),
    scratch_types=[
        pltpu.VMEM((2, CH, CHB), jnp.int32),
        pltpu.VMEM((2, NG * L, D), jnp.float32),
        pltpu.VMEM((2, NG, D), jnp.float32),
        pltpu.VMEM((2, LANES), jnp.int32),
        pltpu.SemaphoreType.DMA,
        pltpu.SemaphoreType.DMA,
        pltpu.SemaphoreType.DMA,
        pltpu.SemaphoreType.DMA,
        pltpu.SemaphoreType.DMA,
        pltpu.SemaphoreType.DMA,
    ],
)(_body)


def kernel(indices, tables):
    idx = indices.astype(jnp.int32).reshape(IDX_ROWS, CHB)
    tab = tables.reshape(F * V, D)
    out = _lookup(idx, tab)
    return out.reshape(B, F, D)


# speedup vs baseline: 1.5675x; 1.0437x over previous
import functools

import jax
import jax.numpy as jnp
from jax import lax
from jax.experimental import pallas as pl
from jax.experimental.pallas import tpu as pltpu
from jax.experimental.pallas import tpu_sc as plsc

F = 26
B = 4096
L = 20
D = 128
V = 100000

_info = plsc.get_sparse_core_info()
NC, NS, LANES = _info.num_cores, _info.num_subcores, _info.num_lanes
NW = NC * NS
BAGS = B * F
BPW = BAGS // NW
NG = 16
NIT = BPW // NG
DCH = D // LANES


def _body(idx_hbm, tab_hbm, out_hbm, idx_v, rows_v, out_v, oidx_v,
          gsem0, gsem1, osem0, osem1, isem0, isem1):
    wid = lax.axis_index("s") * NC + lax.axis_index("c")
    g0w = wid * BPW
    gsems = (gsem0, gsem1)
    osems = (osem0, osem1)
    isems = (isem0, isem1)

    def coords(it):
        g0 = g0w + it * NG
        return g0 // B, g0 % B

    def idx_descr(it, slot):
        f, b0 = coords(it)
        return pltpu.make_async_copy(
            idx_hbm.at[f, pl.ds(b0, NG)],
            idx_v.at[slot],
            isems[slot],
        )

    def start_gathers(it, slot):
        f, _ = coords(it)
        tab_f = tab_hbm.at[pl.ds(f * V, V)]
        for bag in range(NG):
            pltpu.make_async_copy(
                tab_f.at[idx_v.at[slot, bag]],
                rows_v.at[slot, pl.ds(bag * L, L)],
                gsems[slot],
            ).start()

    def drain_gathers(slot):
        pltpu.make_async_copy(
            tab_hbm.at[pl.ds(0, NG * L)],
            rows_v.at[slot],
            gsems[slot],
        ).wait()

    def out_descr(slot):
        return pltpu.make_async_copy(
            out_v.at[slot],
            out_hbm.at[oidx_v.at[slot]],
            osems[slot],
        )

    def stage(it, slot):
        idx_descr(it, slot).wait()
        start_gathers(it, slot)

    def accumulate(it, slot):
        def bag_body(jj, c2):
            base = jj * L
            for c in range(DCH):
                vals = [rows_v[slot, base + l, pl.ds(c * LANES, LANES)]
                        for l in range(L)]
                while len(vals) > 1:
                    nxt = [vals[i] + vals[i + 1]
                           for i in range(0, len(vals) - 1, 2)]
                    if len(vals) % 2:
                        nxt.append(vals[-1])
                    vals = nxt
                out_v[slot, jj, pl.ds(c * LANES, LANES)] = vals[0]
            return c2

        lax.fori_loop(0, NG, bag_body, 0)
        f, b0 = coords(it)
        oidx_v[slot, :] = (b0 * F + f) + F * lax.iota(jnp.int32, LANES)

    idx_descr(0, 0).start()
    stage(0, 0)
    idx_descr(1, 1).start()

    def half(it, s):
        cur = it + s
        ns = 1 - s

        drain_gathers(s)

        @pl.when(cur + 2 < NIT)
        def _():
            idx_descr(cur + 2, s).start()

        @pl.when(cur + 1 < NIT)
        def _():
            stage(cur + 1, ns)

        @pl.when(cur >= 2)
        def _():
            out_descr(s).wait()

        accumulate(cur, s)
        out_descr(s).start()

    def it_body(i, carry):
        half(2 * i, 0)
        half(2 * i, 1)
        return carry

    lax.fori_loop(0, NIT // 2, it_body, 0)
    out_descr(0).wait()
    out_descr(1).wait()


_mesh = plsc.VectorSubcoreMesh(core_axis_name="c", subcore_axis_name="s")

_lookup = functools.partial(
    pl.kernel,
    mesh=_mesh,
    out_type=jax.ShapeDtypeStruct((BAGS, D), jnp.float32),
    scratch_types=[
        pltpu.VMEM((2, NG, L), jnp.int32),
        pltpu.VMEM((2, NG * L, D), jnp.float32),
        pltpu.VMEM((2, NG, D), jnp.float32),
        pltpu.VMEM((2, LANES), jnp.int32),
        pltpu.SemaphoreType.DMA,
        pltpu.SemaphoreType.DMA,
        pltpu.SemaphoreType.DMA,
        pltpu.SemaphoreType.DMA,
        pltpu.SemaphoreType.DMA,
        pltpu.SemaphoreType.DMA,
    ],
)(_body)


def kernel(indices, tables):
    idx = indices.astype(jnp.int32)
    tab = tables.reshape(F * V, D)
    out = _lookup(idx, tab)
    return out.reshape(B, F, D)


# speedup vs baseline: 1.5697x; 1.0014x over previous
import functools

import jax
import jax.numpy as jnp
from jax import lax
from jax.experimental import pallas as pl
from jax.experimental.pallas import tpu as pltpu
from jax.experimental.pallas import tpu_sc as plsc

F = 26
B = 4096
L = 20
D = 128
V = 100000

_info = plsc.get_sparse_core_info()
NC, NS, LANES = _info.num_cores, _info.num_subcores, _info.num_lanes
NW = NC * NS
BAGS = B * F
BPW = BAGS // NW
NG = 16
NIT = BPW // NG
DCH = D // LANES


def _body(idx_hbm, tab_hbm, out_hbm, idx_v, rows_v, out_v, oidx_v,
          gsem0a, gsem0b, gsem1a, gsem1b, osem0, osem1, isem0, isem1):
    wid = lax.axis_index("s") * NC + lax.axis_index("c")
    g0w = wid * BPW
    gsems = ((gsem0a, gsem0b), (gsem1a, gsem1b))
    osems = (osem0, osem1)
    isems = (isem0, isem1)
    NGH = NG // 2

    def coords(it):
        g0 = g0w + it * NG
        return g0 // B, g0 % B

    def idx_descr(it, slot):
        f, b0 = coords(it)
        return pltpu.make_async_copy(
            idx_hbm.at[f, pl.ds(b0, NG)],
            idx_v.at[slot],
            isems[slot],
        )

    def start_gathers(it, slot):
        f, _ = coords(it)
        tab_f = tab_hbm.at[pl.ds(f * V, V)]
        for bag in range(NG):
            pltpu.make_async_copy(
                tab_f.at[idx_v.at[slot, bag]],
                rows_v.at[slot, pl.ds(bag * L, L)],
                gsems[slot][bag // NGH],
            ).start()

    def drain_gathers(slot, h):
        pltpu.make_async_copy(
            tab_hbm.at[pl.ds(0, NGH * L)],
            rows_v.at[slot, pl.ds(h * NGH * L, NGH * L)],
            gsems[slot][h],
        ).wait()

    def out_descr(slot):
        return pltpu.make_async_copy(
            out_v.at[slot],
            out_hbm.at[oidx_v.at[slot]],
            osems[slot],
        )

    def stage(it, slot):
        idx_descr(it, slot).wait()
        start_gathers(it, slot)

    def pool_half(slot, h):
        def bag_body(jj, c2):
            base = jj * L
            for c in range(DCH):
                vals = [rows_v[slot, base + l, pl.ds(c * LANES, LANES)]
                        for l in range(L)]
                while len(vals) > 1:
                    nxt = [vals[i] + vals[i + 1]
                           for i in range(0, len(vals) - 1, 2)]
                    if len(vals) % 2:
                        nxt.append(vals[-1])
                    vals = nxt
                out_v[slot, jj, pl.ds(c * LANES, LANES)] = vals[0]
            return c2

        lax.fori_loop(h * NGH, (h + 1) * NGH, bag_body, 0)

    def set_out_ids(it, slot):
        f, b0 = coords(it)
        oidx_v[slot, :] = (b0 * F + f) + F * lax.iota(jnp.int32, LANES)

    idx_descr(0, 0).start()
    stage(0, 0)
    idx_descr(1, 1).start()

    def half(it, s):
        cur = it + s
        ns = 1 - s

        drain_gathers(s, 0)

        @pl.when(cur + 2 < NIT)
        def _():
            idx_descr(cur + 2, s).start()

        @pl.when(cur + 1 < NIT)
        def _():
            stage(cur + 1, ns)

        @pl.when(cur >= 2)
        def _():
            out_descr(s).wait()

        pool_half(s, 0)
        drain_gathers(s, 1)
        pool_half(s, 1)
        set_out_ids(cur, s)
        out_descr(s).start()

    def it_body(i, carry):
        half(2 * i, 0)
        half(2 * i, 1)
        return carry

    lax.fori_loop(0, NIT // 2, it_body, 0)
    out_descr(0).wait()
    out_descr(1).wait()


_mesh = plsc.VectorSubcoreMesh(core_axis_name="c", subcore_axis_name="s")

_lookup = functools.partial(
    pl.kernel,
    mesh=_mesh,
    out_type=jax.ShapeDtypeStruct((BAGS, D), jnp.float32),
    scratch_types=[
        pltpu.VMEM((2, NG, L), jnp.int32),
        pltpu.VMEM((2, NG * L, D), jnp.float32),
        pltpu.VMEM((2, NG, D), jnp.float32),
        pltpu.VMEM((2, LANES), jnp.int32),
    ] + [pltpu.SemaphoreType.DMA] * 8,
)(_body)


def kernel(indices, tables):
    idx = indices.astype(jnp.int32)
    tab = tables.reshape(F * V, D)
    out = _lookup(idx, tab)
    return out.reshape(B, F, D)


# speedup vs baseline: 1.7155x; 1.0929x over previous
import functools

import jax
import jax.numpy as jnp
from jax import lax
from jax.experimental import pallas as pl
from jax.experimental.pallas import tpu as pltpu
from jax.experimental.pallas import tpu_sc as plsc

F = 26
B = 4096
L = 20
D = 128
V = 100000

_info = plsc.get_sparse_core_info()
NC, NS, LANES = _info.num_cores, _info.num_subcores, _info.num_lanes
NW = NC * NS
BAGS = B * F
BPW = BAGS // NW
NG = 16
NIT = BPW // NG
DCH = D // LANES


def _body(idx_hbm, tab_hbm, out_hbm, idx_v, rows_v, out_v,
          gsem0a, gsem0b, gsem1a, gsem1b, osem0, osem1, isem0, isem1):
    wid = lax.axis_index("s") * NC + lax.axis_index("c")
    g0w = wid * BPW
    gsems = ((gsem0a, gsem0b), (gsem1a, gsem1b))
    osems = (osem0, osem1)
    isems = (isem0, isem1)
    NGH = NG // 2

    def coords(it):
        g0 = g0w + it * NG
        return g0 // B, g0 % B

    def idx_descr(it, slot):
        f, b0 = coords(it)
        return pltpu.make_async_copy(
            idx_hbm.at[f, pl.ds(b0, NG)],
            idx_v.at[slot],
            isems[slot],
        )

    def start_gathers(it, slot):
        f, _ = coords(it)
        tab_f = tab_hbm.at[pl.ds(f * V, V)]
        for bag in range(NG):
            pltpu.make_async_copy(
                tab_f.at[idx_v.at[slot, bag]],
                rows_v.at[slot, pl.ds(bag * L, L)],
                gsems[slot][bag // NGH],
            ).start()

    def drain_gathers(slot, h):
        pltpu.make_async_copy(
            tab_hbm.at[pl.ds(0, NGH * L)],
            rows_v.at[slot, pl.ds(h * NGH * L, NGH * L)],
            gsems[slot][h],
        ).wait()

    def out_descr(it, slot):
        f, b0 = coords(it)
        return pltpu.make_async_copy(
            out_v.at[slot],
            out_hbm.at[pl.ds(b0, NG), f],
            osems[slot],
        )

    def stage(it, slot):
        idx_descr(it, slot).wait()
        start_gathers(it, slot)

    def pool_half(slot, h):
        def bag_body(jj, c2):
            base = jj * L
            for c in range(DCH):
                vals = [rows_v[slot, base + l, pl.ds(c * LANES, LANES)]
                        for l in range(L)]
                while len(vals) > 1:
                    nxt = [vals[i] + vals[i + 1]
                           for i in range(0, len(vals) - 1, 2)]
                    if len(vals) % 2:
                        nxt.append(vals[-1])
                    vals = nxt
                out_v[slot, jj, pl.ds(c * LANES, LANES)] = vals[0]
            return c2

        lax.fori_loop(h * NGH, (h + 1) * NGH, bag_body, 0)

    idx_descr(0, 0).start()
    stage(0, 0)
    idx_descr(1, 1).start()

    def half(it, s):
        cur = it + s
        ns = 1 - s

        drain_gathers(s, 0)

        @pl.when(cur + 2 < NIT)
        def _():
            idx_descr(cur + 2, s).start()

        @pl.when(cur + 1 < NIT)
        def _():
            stage(cur + 1, ns)

        @pl.when(cur >= 2)
        def _():
            out_descr(cur - 2, s).wait()

        pool_half(s, 0)
        drain_gathers(s, 1)
        pool_half(s, 1)
        out_descr(cur, s).start()

    def it_body(i, carry):
        half(2 * i, 0)
        half(2 * i, 1)
        return carry

    lax.fori_loop(0, NIT // 2, it_body, 0)
    out_descr(NIT - 2, 0).wait()
    out_descr(NIT - 1, 1).wait()


_mesh = plsc.VectorSubcoreMesh(core_axis_name="c", subcore_axis_name="s")

_lookup = functools.partial(
    pl.kernel,
    mesh=_mesh,
    out_type=jax.ShapeDtypeStruct((B, F, D), jnp.float32),
    scratch_types=[
        pltpu.VMEM((2, NG, L), jnp.int32),
        pltpu.VMEM((2, NG * L, D), jnp.float32),
        pltpu.VMEM((2, NG, D), jnp.float32),
    ] + [pltpu.SemaphoreType.DMA] * 8,
)(_body)


def kernel(indices, tables):
    idx = indices.astype(jnp.int32)
    tab = tables.reshape(F * V, D)
    return _lookup(idx, tab)
